# baseline (device time: 79270 ns/iter reference)
import jax
import jax.numpy as jnp
from jax import lax
from jax.experimental import pallas as pl
from jax.experimental.pallas import tpu as pltpu

N_DEV = 8
N_SLOT = 3
FP8 = jnp.float8_e4m3fn


def kernel(x, w_mat, scale_x, scale_w):
    M, K_sh = x.shape
    K, N = w_mat.shape
    M_blk = M // N_DEV
    HN = N // 2
    N_STEP = 2 * N_DEV

    def body(x_ref, w_ref, sx_ref, sw_ref, out_ref,
             xq_ref, comm_ref, wq_ref, xs_ref, ob_ref,
             send_sems, recv_sems, w_sems, x_sem, o_sems):
        my = lax.axis_index("i")

        def w_copy(t, slot):
            d = lax.rem(t, N_DEV)
            nh = t // N_DEV
            j = lax.rem(my - d + N_DEV, N_DEV)
            return pltpu.make_async_copy(
                w_ref.at[pl.ds(j * K_sh, K_sh), pl.ds(nh * HN, HN)],
                wq_ref.at[slot],
                w_sems.at[slot],
            )

        for t in range(N_SLOT):
            w_copy(t, t).start()
        x_copy = pltpu.make_async_copy(x_ref, xs_ref, x_sem)
        x_copy.start()

        x_copy.wait()
        xq_ref[...] = xs_ref[...].astype(FP8)
        comm_ref[N_DEV - 1, :, :] = xq_ref[pl.ds(my * M_blk, M_blk), :]

        barrier = pltpu.get_barrier_semaphore()

        def bar(d, c):
            peer = lax.rem(my + d, N_DEV)
            pl.semaphore_signal(barrier, inc=1, device_id=(peer,),
                                device_id_type=pl.DeviceIdType.MESH)
            return c

        lax.fori_loop(1, N_DEV, bar, 0)
        pl.semaphore_wait(barrier, N_DEV - 1)

        def a2a(d, c):
            peer = lax.rem(my + d, N_DEV)
            pltpu.make_async_remote_copy(
                src_ref=xq_ref.at[pl.ds(peer * M_blk, M_blk), :],
                dst_ref=comm_ref.at[d - 1],
                send_sem=send_sems.at[d - 1],
                recv_sem=recv_sems.at[d - 1],
                device_id=(peer,),
                device_id_type=pl.DeviceIdType.MESH,
            ).start()
            return c

        lax.fori_loop(1, N_DEV, a2a, 0)

        scale = sx_ref[0] * sw_ref[0]

        def out_dma(nh):
            return pltpu.make_async_copy(
                ob_ref.at[nh],
                out_ref.at[:, pl.ds(nh * HN, HN)],
                o_sems.at[nh],
            )

        def step(t, c):
            d = lax.rem(t, N_DEV)
            nh = t // N_DEV
            slot = lax.rem(t, N_SLOT)
            lhs_idx = jnp.where(d == 0, N_DEV - 1, d - 1)

            w_copy(t, slot).wait()
            wblk = wq_ref[slot].astype(FP8)

            @pl.when(t + N_SLOT < N_STEP)
            def _():
                w_copy(t + N_SLOT, slot).start()

            @pl.when(jnp.logical_and(t > 0, t < N_DEV))
            def _():
                pltpu.make_async_remote_copy(
                    src_ref=xq_ref.at[pl.ds(0, M_blk), :],
                    dst_ref=comm_ref.at[d - 1],
                    send_sem=send_sems.at[0],
                    recv_sem=recv_sems.at[d - 1],
                    device_id=(my,),
                    device_id_type=pl.DeviceIdType.MESH,
                ).wait_recv()

            part = lax.dot_general(
                comm_ref[lhs_idx], wblk,
                (((1,), (0,)), ((), ())), preferred_element_type=jnp.float32)

            @pl.when(d == 0)
            def _():
                ob_ref[nh] = part

            @pl.when(d > 0)
            def _():
                ob_ref[nh] = ob_ref[nh] + part

            @pl.when(d == N_DEV - 1)
            def _():
                ob_ref[nh] = ob_ref[nh] * scale
                out_dma(nh).start()

            return c

        lax.fori_loop(0, N_STEP, step, 0)

        out_dma(0).wait()
        out_dma(1).wait()

        def drain(d, c):
            pltpu.make_async_remote_copy(
                src_ref=xq_ref.at[pl.ds(0, M_blk), :],
                dst_ref=comm_ref.at[0],
                send_sem=send_sems.at[d - 1],
                recv_sem=recv_sems.at[0],
                device_id=(my,),
                device_id_type=pl.DeviceIdType.MESH,
            ).wait_send()
            return c

        lax.fori_loop(1, N_DEV, drain, 0)

    return pl.pallas_call(
        body,
        out_shape=jax.ShapeDtypeStruct((M_blk, N), jnp.float32),
        in_specs=[
            pl.BlockSpec(memory_space=pl.ANY),
            pl.BlockSpec(memory_space=pl.ANY),
            pl.BlockSpec(memory_space=pltpu.SMEM),
            pl.BlockSpec(memory_space=pltpu.SMEM),
        ],
        out_specs=pl.BlockSpec(memory_space=pl.ANY),
        scratch_shapes=[
            pltpu.VMEM((M, K_sh), FP8),
            pltpu.VMEM((N_DEV, M_blk, K_sh), FP8),
            pltpu.VMEM((N_SLOT, K_sh, HN), jnp.float32),
            pltpu.VMEM((M, K_sh), jnp.float32),
            pltpu.VMEM((2, M_blk, HN), jnp.float32),
            pltpu.SemaphoreType.DMA((N_DEV - 1,)),
            pltpu.SemaphoreType.DMA((N_DEV - 1,)),
            pltpu.SemaphoreType.DMA((N_SLOT,)),
            pltpu.SemaphoreType.DMA(()),
            pltpu.SemaphoreType.DMA((2,)),
        ],
        compiler_params=pltpu.CompilerParams(
            collective_id=0, vmem_limit_bytes=100 * 1024 * 1024,
        ),
    )(x, w_mat, scale_x, scale_w)


# device time: 68002 ns/iter; 1.1657x vs baseline; 1.1657x over previous
import jax
import jax.numpy as jnp
from jax import lax
from jax.experimental import pallas as pl
from jax.experimental.pallas import tpu as pltpu

N_DEV = 8
FP8 = jnp.float8_e4m3fn


def kernel(x, w_mat, scale_x, scale_w):
    M, K_sh = x.shape
    K, N = w_mat.shape
    M_blk = M // N_DEV

    def body(x_ref, w_ref, sx_ref, sw_ref, out_ref,
             xq_ref, comm_ref, wq_ref, xs_ref,
             send_sems, recv_sems, w_sems, x_sem):
        my = lax.axis_index("i")

        def w_copy(d, slot):
            j = lax.rem(my - d + N_DEV, N_DEV)
            return pltpu.make_async_copy(
                w_ref.at[pl.ds(j * K_sh, K_sh), :],
                wq_ref.at[slot],
                w_sems.at[slot],
            )

        w_copy(0, 0).start()
        w_copy(1, 1).start()
        x_copy = pltpu.make_async_copy(x_ref, xs_ref, x_sem)
        x_copy.start()

        x_copy.wait()
        xq_ref[...] = xs_ref[...].astype(FP8)

        barrier = pltpu.get_barrier_semaphore()

        def bar(d, c):
            peer = lax.rem(my + d, N_DEV)
            pl.semaphore_signal(barrier, inc=1, device_id=(peer,),
                                device_id_type=pl.DeviceIdType.MESH)
            return c

        lax.fori_loop(1, N_DEV, bar, 0)
        pl.semaphore_wait(barrier, N_DEV - 1)

        def a2a(d, c):
            peer = lax.rem(my + d, N_DEV)
            pltpu.make_async_remote_copy(
                src_ref=xq_ref.at[pl.ds(peer * M_blk, M_blk), :],
                dst_ref=comm_ref.at[d - 1],
                send_sem=send_sems.at[d - 1],
                recv_sem=recv_sems.at[d - 1],
                device_id=(peer,),
                device_id_type=pl.DeviceIdType.MESH,
            ).start()
            return c

        lax.fori_loop(1, N_DEV, a2a, 0)

        scale = sx_ref[0] * sw_ref[0]

        w_copy(0, 0).wait()
        out_ref[...] = lax.dot_general(
            xq_ref[pl.ds(my * M_blk, M_blk), :], wq_ref[0].astype(FP8),
            (((1,), (0,)), ((), ())), preferred_element_type=jnp.float32)

        def step(d, c):
            slot = lax.rem(d, 2)

            @pl.when(d + 1 < N_DEV)
            def _():
                w_copy(d + 1, 1 - slot).start()

            w_copy(d, slot).wait()
            pltpu.make_async_remote_copy(
                src_ref=xq_ref.at[pl.ds(0, M_blk), :],
                dst_ref=comm_ref.at[d - 1],
                send_sem=send_sems.at[0],
                recv_sem=recv_sems.at[d - 1],
                device_id=(my,),
                device_id_type=pl.DeviceIdType.MESH,
            ).wait_recv()
            out_ref[...] = out_ref[...] + lax.dot_general(
                comm_ref[d - 1], wq_ref[slot].astype(FP8),
                (((1,), (0,)), ((), ())), preferred_element_type=jnp.float32)
            return c

        lax.fori_loop(1, N_DEV, step, 0)

        out_ref[...] = out_ref[...] * scale

        def drain(d, c):
            pltpu.make_async_remote_copy(
                src_ref=xq_ref.at[pl.ds(0, M_blk), :],
                dst_ref=comm_ref.at[0],
                send_sem=send_sems.at[d - 1],
                recv_sem=recv_sems.at[0],
                device_id=(my,),
                device_id_type=pl.DeviceIdType.MESH,
            ).wait_send()
            return c

        lax.fori_loop(1, N_DEV, drain, 0)

    return pl.pallas_call(
        body,
        out_shape=jax.ShapeDtypeStruct((M_blk, N), jnp.float32),
        in_specs=[
            pl.BlockSpec(memory_space=pl.ANY),
            pl.BlockSpec(memory_space=pl.ANY),
            pl.BlockSpec(memory_space=pltpu.SMEM),
            pl.BlockSpec(memory_space=pltpu.SMEM),
        ],
        out_specs=pl.BlockSpec(memory_space=pltpu.VMEM),
        scratch_shapes=[
            pltpu.VMEM((M, K_sh), FP8),
            pltpu.VMEM((N_DEV - 1, M_blk, K_sh), FP8),
            pltpu.VMEM((2, K_sh, N), jnp.float32),
            pltpu.VMEM((M, K_sh), jnp.float32),
            pltpu.SemaphoreType.DMA((N_DEV - 1,)),
            pltpu.SemaphoreType.DMA((N_DEV - 1,)),
            pltpu.SemaphoreType.DMA((2,)),
            pltpu.SemaphoreType.DMA(()),
        ],
        compiler_params=pltpu.CompilerParams(
            collective_id=0, vmem_limit_bytes=100 * 1024 * 1024,
        ),
    )(x, w_mat, scale_x, scale_w)
